# baseline (speedup 1.0000x reference)
import jax
import jax.numpy as jnp
from jax.experimental import pallas as pl

DEPTH = 1000
N_ROWS = 4096 * 26
FLAT = N_ROWS * DEPTH
FILL_COLS = 1024
FILL_ROWS = FLAT // FILL_COLS
FILL_BLOCK_ROWS = 2000


def _zero_block(o_ref):
    o_ref[...] = jnp.zeros_like(o_ref)


def kernel(inputs):
    out = pl.pallas_call(
        _zero_block,
        grid=(FILL_ROWS // FILL_BLOCK_ROWS,),
        out_specs=pl.BlockSpec((FILL_BLOCK_ROWS, FILL_COLS), lambda i: (i, 0)),
        out_shape=jax.ShapeDtypeStruct((FILL_ROWS, FILL_COLS), jnp.float32),
    )()
    return out.reshape(4096, 26, DEPTH)


# speedup vs baseline: 2.7355x; 2.7355x over previous
import jax
import jax.numpy as jnp
from jax import lax
from jax.experimental import pallas as pl

DEPTH = 1000
B0 = 32
GRID = 4096 // B0


def _onehot_block(idx_ref, out_ref):
    idx = idx_ref[...]
    iota = lax.broadcasted_iota(jnp.int32, out_ref.shape, 2)
    out_ref[...] = jnp.where(idx[:, :, None] == iota, 1.0, 0.0)


def kernel(inputs):
    return pl.pallas_call(
        _onehot_block,
        grid=(GRID,),
        in_specs=[pl.BlockSpec((B0, 26), lambda i: (i, 0))],
        out_specs=pl.BlockSpec((B0, 26, DEPTH), lambda i: (i, 0, 0)),
        out_shape=jax.ShapeDtypeStruct((4096, 26, DEPTH), jnp.float32),
    )(inputs)
